# baseline (speedup 1.0000x reference)
import functools

import jax
import jax.numpy as jnp
from jax import lax
from jax.experimental import pallas as pl
from jax.experimental.pallas import tpu as pltpu
from jax.experimental.pallas import tpu_sc as plsc

_G = 256
_D = 128
_GSD = 128
_H = 2 * _GSD
_TILE = 8192
_NPAD = 106496
_NW = 32
_RPW = _NPAD // _NW
_CHUNK = 416


def _mlp1(x_ref, w1_ref, b1_ref, g_ref, *, n_rows):
    i = pl.program_id(0)
    x = x_ref[...].astype(jnp.bfloat16)
    h = jax.lax.dot_general(x, w1_ref[...].astype(jnp.bfloat16),
                            (((1,), (1,)), ((), ())),
                            preferred_element_type=jnp.float32)
    h = h + b1_ref[...]
    g = h[:, _GSD:] * jax.nn.sigmoid(h[:, :_GSD])
    row = i * _TILE + jax.lax.broadcasted_iota(jnp.int32, (_TILE, 1), 0)
    g_ref[...] = jnp.where(row < n_rows, g, 0.0)


def _mlp2(p_ref, w2_ref, b2_ref, out_ref):
    p = p_ref[0] + p_ref[1]
    out = jax.lax.dot_general(p, w2_ref[...], (((1,), (1,)), ((), ())),
                              preferred_element_type=jnp.float32)
    out_ref[...] = out + b2_ref[...]


def _sc_segsum(g_hbm, idx_hbm, out_hbm, rows_v, idx_v, zeros_v, acc_sh):
    cid = lax.axis_index("c")
    sid = lax.axis_index("s")
    wid = cid * 16 + sid
    base = wid * _RPW

    @pl.when(sid == 0)
    def _zero():
        zeros_v[...] = jnp.zeros_like(zeros_v)
        pltpu.sync_copy(zeros_v, acc_sh)

    plsc.subcore_barrier()

    for c in range(_RPW // _CHUNK):
        off = base + c * _CHUNK
        pltpu.sync_copy(g_hbm.at[pl.ds(off, _CHUNK)], rows_v)
        pltpu.sync_copy(idx_hbm.at[pl.ds(off, _CHUNK)], idx_v)
        pltpu.sync_copy(rows_v, acc_sh.at[idx_v], add=True)

    plsc.subcore_barrier()

    @pl.when(sid == 0)
    def _drain():
        pltpu.sync_copy(acc_sh, out_hbm.at[cid])


def kernel(node_states, graph_idx, n_graphs, W1, b1, W2, b2):
    n = node_states.shape[0]
    nsteps = pl.cdiv(n, _TILE)
    idx = jnp.minimum(graph_idx.astype(jnp.int32), _G - 1)
    idx = jnp.pad(idx, (0, _NPAD - n), constant_values=_G - 1)

    g = pl.pallas_call(
        functools.partial(_mlp1, n_rows=n),
        grid=(nsteps,),
        in_specs=[
            pl.BlockSpec((_TILE, _D), lambda i: (i, 0)),
            pl.BlockSpec((_H, _D), lambda i: (0, 0)),
            pl.BlockSpec((1, _H), lambda i: (0, 0)),
        ],
        out_specs=pl.BlockSpec((_TILE, _GSD), lambda i: (i, 0)),
        out_shape=jax.ShapeDtypeStruct((_NPAD, _GSD), jnp.float32),
        compiler_params=pltpu.CompilerParams(
            dimension_semantics=("arbitrary",)),
    )(node_states, W1, b1.reshape(1, _H))

    mesh = plsc.VectorSubcoreMesh(core_axis_name="c", subcore_axis_name="s")
    partials = pl.kernel(
        _sc_segsum,
        mesh=mesh,
        out_type=jax.ShapeDtypeStruct((2, _G, _GSD), jnp.float32),
        scratch_types=[
            pltpu.VMEM((_CHUNK, _GSD), jnp.float32),
            pltpu.VMEM((_CHUNK,), jnp.int32),
            pltpu.VMEM((_G, _GSD), jnp.float32),
            pltpu.VMEM_SHARED((_G, _GSD), jnp.float32),
        ],
    )(g, idx)

    out = pl.pallas_call(
        _mlp2,
        grid=(1,),
        in_specs=[
            pl.BlockSpec((2, _G, _GSD), lambda i: (0, 0, 0)),
            pl.BlockSpec((_GSD, _GSD), lambda i: (0, 0)),
            pl.BlockSpec((1, _GSD), lambda i: (0, 0)),
        ],
        out_specs=pl.BlockSpec((_G, _GSD), lambda i: (0, 0)),
        out_shape=jax.ShapeDtypeStruct((_G, _GSD), jnp.float32),
    )(partials, W2, b2.reshape(1, _GSD))
    return out


# speedup vs baseline: 2.7823x; 2.7823x over previous
import functools

import jax
import jax.numpy as jnp
from jax.experimental import pallas as pl
from jax.experimental.pallas import tpu as pltpu

_G = 256
_D = 128
_GSD = 128
_H = 2 * _GSD
_TILE = 8192


def _fused(x_ref, idx_ref, w1_ref, b1_ref, w2_ref, b2_ref, out_ref, acc_ref,
           *, n_rows):
    i = pl.program_id(0)
    nsteps = pl.num_programs(0)

    @pl.when(i == 0)
    def _init():
        acc_ref[...] = jnp.zeros_like(acc_ref)

    x = x_ref[...].astype(jnp.bfloat16)
    h = jax.lax.dot_general(x, w1_ref[...].astype(jnp.bfloat16),
                            (((1,), (1,)), ((), ())),
                            preferred_element_type=jnp.float32)
    h = h + b1_ref[...]
    g = h[:, _GSD:] * jax.nn.sigmoid(h[:, :_GSD])

    row = i * _TILE + jax.lax.broadcasted_iota(jnp.int32, (_TILE, 1), 0)
    g = jnp.where(row < n_rows, g, 0.0)

    idx = idx_ref[0, 0, :]
    onehot = (jax.lax.broadcasted_iota(jnp.int32, (_G, _TILE), 0)
              == idx[None, :]).astype(jnp.bfloat16)
    acc_ref[...] += jax.lax.dot_general(onehot, g.astype(jnp.bfloat16),
                                        (((1,), (0,)), ((), ())),
                                        preferred_element_type=jnp.float32)

    @pl.when(i == nsteps - 1)
    def _finish():
        out = jax.lax.dot_general(acc_ref[...], w2_ref[...],
                                  (((1,), (1,)), ((), ())),
                                  preferred_element_type=jnp.float32)
        out_ref[...] = out + b2_ref[...]


def kernel(node_states, graph_idx, n_graphs, W1, b1, W2, b2):
    n = node_states.shape[0]
    nsteps = pl.cdiv(n, _TILE)
    npad = nsteps * _TILE
    idx = jnp.minimum(graph_idx.astype(jnp.int32), _G - 1)
    idx = jnp.pad(idx, (0, npad - n), constant_values=_G)
    idx3 = idx.reshape(nsteps, 1, _TILE)

    out = pl.pallas_call(
        functools.partial(_fused, n_rows=n),
        grid=(nsteps,),
        in_specs=[
            pl.BlockSpec((_TILE, _D), lambda i: (i, 0)),
            pl.BlockSpec((1, 1, _TILE), lambda i: (i, 0, 0)),
            pl.BlockSpec((_H, _D), lambda i: (0, 0)),
            pl.BlockSpec((1, _H), lambda i: (0, 0)),
            pl.BlockSpec((_GSD, _GSD), lambda i: (0, 0)),
            pl.BlockSpec((1, _GSD), lambda i: (0, 0)),
        ],
        out_specs=pl.BlockSpec((_G, _GSD), lambda i: (0, 0)),
        out_shape=jax.ShapeDtypeStruct((_G, _GSD), jnp.float32),
        scratch_shapes=[pltpu.VMEM((_G, _GSD), jnp.float32)],
        compiler_params=pltpu.CompilerParams(
            dimension_semantics=("arbitrary",)),
    )(node_states, idx3, W1, b1.reshape(1, _H), W2, b2.reshape(1, _GSD))
    return out
